# baseline (speedup 1.0000x reference)
import functools

import jax
import jax.numpy as jnp
from jax import lax
from jax.experimental import pallas as pl
from jax.experimental.pallas import tpu as pltpu
from jax.experimental.pallas import tpu_sc as plsc

N = 50000
D_IN = 128
H = 32

E = 800000
LANES = 256
NC = 2
NS = 16
RPT = 196
EROWS = RPT * NS
EPAD = EROWS * LANES - E

NACC = 50048
OPT = NACC // NS

P = 4
PW = H // P
CH = 1
NCH = RPT // CH

BLK = 2000
GRID = N // BLK


def _prep_edges(ei, r):
    src = jnp.concatenate(
        [ei[0] * 16 + 4 * r, jnp.full((EPAD,), 4 * r, jnp.int32)])
    dst = jnp.concatenate([ei[1], jnp.full((EPAD,), N, jnp.int32)])
    return src.reshape(EROWS, LANES), dst.reshape(EROWS, LANES)


@functools.cache
def _sc_mesh():
    return plsc.VectorSubcoreMesh(
        core_axis_name="c", subcore_axis_name="s",
        num_cores=NC, num_subcores=NS)


def _deg_body(edges, z8, o8, out, idx_v, ones_v, hist, sem):
    c = lax.axis_index("c")
    s = lax.axis_index("s")

    pltpu.sync_copy(o8, ones_v)

    def one_rel(r):
        pltpu.sync_copy(z8, hist.at[pl.ds(s * OPT, OPT)])
        pltpu.sync_copy(edges.at[4 + r].at[pl.ds(s * RPT, RPT)], idx_v)
        plsc.subcore_barrier()

        @pl.loop(0, RPT)
        def _(k):
            pltpu.async_copy(ones_v, hist.at[idx_v.at[k]], sem, add=True)

        @pl.loop(0, RPT)
        def _(k):
            pltpu.make_async_copy(ones_v, hist.at[idx_v.at[k]], sem).wait()

        plsc.subcore_barrier()
        pltpu.sync_copy(hist.at[pl.ds(s * OPT, OPT)],
                        out.at[pl.ds(s * OPT, OPT), pl.ds(8 * r, PW)])
        plsc.subcore_barrier()

    @pl.when(c == 0)
    def _():
        one_rel(0)
        one_rel(1)

    @pl.when(c == 1)
    def _():
        one_rel(2)
        one_rel(3)


@functools.cache
def _deg_kernel():
    return functools.partial(
        pl.kernel,
        out_type=jax.ShapeDtypeStruct((NACC, H), jnp.float32),
        mesh=_sc_mesh(),
        compiler_params=pltpu.CompilerParams(use_tc_tiling_on_sc=False),
        scratch_types=[
            pltpu.VMEM((RPT, LANES), jnp.int32),
            pltpu.VMEM((LANES, PW), jnp.float32),
            pltpu.VMEM_SHARED((NACC, PW), jnp.float32),
            pltpu.SemaphoreType.DMA,
        ],
    )(_deg_body)


def _run_deg(*args):
    return _deg_kernel()(*args)


def _agg_body(y8, edges, z8, out, isrc, idst, rows_a, rows_b, acc,
              gsem, ssem):
    c = lax.axis_index("c")
    s = lax.axis_index("s")

    def fire_g(yp, c0, buf):
        for i in range(CH):
            pltpu.async_copy(yp.at[isrc.at[c0 * CH + i]], buf.at[i], gsem)

    def wait_g(yp, c0, buf):
        for i in range(CH):
            pltpu.make_async_copy(
                yp.at[isrc.at[c0 * CH + i]], buf.at[i], gsem).wait()

    def fire_s(c0, buf):
        for i in range(CH):
            pltpu.async_copy(
                buf.at[i], acc.at[idst.at[c0 * CH + i]], ssem, add=True)

    def wait_s(c0, buf):
        for i in range(CH):
            pltpu.make_async_copy(
                buf.at[i], acc.at[idst.at[c0 * CH + i]], ssem).wait()

    def one_rel(r):
        pltpu.sync_copy(edges.at[r].at[pl.ds(s * RPT, RPT)], isrc)
        pltpu.sync_copy(edges.at[4 + r].at[pl.ds(s * RPT, RPT)], idst)
        for p in range(P):
            yp = y8.at[pl.ds(p, 16 * N)]
            pltpu.sync_copy(z8, acc.at[pl.ds(s * OPT, OPT)])
            plsc.subcore_barrier()

            fire_g(yp, 0, rows_a)

            @pl.loop(0, NCH, step=2)
            def _(c0):
                fire_g(yp, c0 + 1, rows_b)
                wait_g(yp, c0, rows_a)
                fire_s(c0, rows_a)
                wait_s(c0, rows_a)

                @pl.when(c0 + 2 < NCH)
                def _():
                    fire_g(yp, c0 + 2, rows_a)

                wait_g(yp, c0 + 1, rows_b)
                fire_s(c0 + 1, rows_b)
                wait_s(c0 + 1, rows_b)

            plsc.subcore_barrier()
            pltpu.sync_copy(
                acc.at[pl.ds(s * OPT, OPT)],
                out.at[pl.ds(s * OPT, OPT), pl.ds(32 * r + 8 * p, PW)])
            plsc.subcore_barrier()

    @pl.when(c == 0)
    def _():
        one_rel(0)
        one_rel(1)

    @pl.when(c == 1)
    def _():
        one_rel(2)
        one_rel(3)


@functools.cache
def _agg_kernel():
    return functools.partial(
        pl.kernel,
        out_type=jax.ShapeDtypeStruct((NACC, 128), jnp.float32),
        mesh=_sc_mesh(),
        compiler_params=pltpu.CompilerParams(use_tc_tiling_on_sc=False),
        scratch_types=[
            pltpu.VMEM((RPT, LANES), jnp.int32),
            pltpu.VMEM((RPT, LANES), jnp.int32),
            pltpu.VMEM((CH, LANES, PW), jnp.float32),
            pltpu.VMEM((CH, LANES, PW), jnp.float32),
            pltpu.VMEM_SHARED((NACC, PW), jnp.float32),
            pltpu.SemaphoreType.DMA,
            pltpu.SemaphoreType.DMA,
        ],
    )(_agg_body)


def _run_agg(*args):
    return _agg_kernel()(*args)


def _dis(deg_blk, r):
    return lax.rsqrt(deg_blk[:, 8 * r:8 * r + 1] + 1.0)


def _k1_body(x, g, w0, w1, w2, w3, yL):
    xb = x[...]
    gb = g[...]
    parts = []
    for r, w in enumerate((w0, w1, w2, w3)):
        parts.append(jnp.dot(xb, w[...],
                             preferred_element_type=jnp.float32) * _dis(gb, r))
    yL[...] = jnp.concatenate(parts, axis=1)


def _k2_body(aL, yL, g, b0, b1, b2, b3, w0, w1, w2, w3, oL):
    ab = aL[...]
    yb = yL[...]
    gb = g[...]
    ds_ = [_dis(gb, r) for r in range(4)]
    h = jnp.zeros((BLK, H), jnp.float32)
    for r, b in enumerate((b0, b1, b2, b3)):
        h = h + ds_[r] * (ab[:, 32 * r:32 * r + 32]
                          + yb[:, 32 * r:32 * r + 32]) + b[...]
    h = jnp.maximum(h, 0.0)
    parts = []
    for r, w in enumerate((w0, w1, w2, w3)):
        parts.append(jnp.dot(h, w[...],
                             preferred_element_type=jnp.float32) * ds_[r])
    oL[...] = jnp.concatenate(parts, axis=1)


def _k3_body(aL, yL, g, b0, b1, b2, b3, h2, stats):
    i = pl.program_id(0)
    ab = aL[...]
    yb = yL[...]
    gb = g[...]
    h = jnp.zeros((BLK, H), jnp.float32)
    for r, b in enumerate((b0, b1, b2, b3)):
        h = h + _dis(gb, r) * (ab[:, 32 * r:32 * r + 32]
                               + yb[:, 32 * r:32 * r + 32]) + b[...]
    h = jnp.maximum(h, 0.0)
    h2[...] = h

    part = jnp.concatenate(
        [jnp.sum(h, axis=0, keepdims=True),
         jnp.sum(h * h, axis=0, keepdims=True),
         jnp.zeros((6, H), jnp.float32)], axis=0)

    @pl.when(i == 0)
    def _():
        stats[...] = jnp.zeros((8, H), jnp.float32)

    stats[...] += part


def _k4_body(h2, mean, scinv, bnb, w1, b1, w2, b2, out):
    hb = (h2[...] - mean[...]) * scinv[...] + bnb[...]
    z = jnp.maximum(
        jnp.dot(hb, w1[...], preferred_element_type=jnp.float32) + b1[...],
        0.0)
    out[...] = jnp.dot(z, w2[...], preferred_element_type=jnp.float32) + b2[...]


def _rowspec(cols):
    return pl.BlockSpec((BLK, cols), lambda i: (i, 0))


def _fullspec(r, cols):
    return pl.BlockSpec((r, cols), lambda i: (0, 0))


def kernel(x, ei_fd, ei_fault, ei_rock, ei_geo, W1_fd, b1_fd, W2_fd, b2_fd,
           W1_fault, b1_fault, W2_fault, b2_fault, W1_rock, b1_rock, W2_rock,
           b2_rock, W1_geo, b1_geo, W2_geo, b2_geo, bn_g, bn_b, lin1_W,
           lin1_b, lin2_W, lin2_b):
    eis = [ei_fd, ei_fault, ei_rock, ei_geo]
    W1s = [W1_fd, W1_fault, W1_rock, W1_geo]
    b1s = [b1_fd.reshape(1, H), b1_fault.reshape(1, H),
           b1_rock.reshape(1, H), b1_geo.reshape(1, H)]
    W2s = [W2_fd, W2_fault, W2_rock, W2_geo]
    b2s = [b2_fd.reshape(1, H), b2_fault.reshape(1, H),
           b2_rock.reshape(1, H), b2_geo.reshape(1, H)]

    srcs, dsts = [], []
    for r, ei in enumerate(eis):
        s2, d2 = _prep_edges(ei, r)
        srcs.append(s2)
        dsts.append(d2)
    edges = jnp.stack(srcs + dsts)

    z8 = jnp.zeros((OPT, PW), jnp.float32)
    o8 = jnp.ones((LANES, PW), jnp.float32)

    degs = _run_deg(edges, z8, o8)

    y1 = pl.pallas_call(
        _k1_body,
        grid=(GRID,),
        in_specs=[_rowspec(D_IN), _rowspec(H)] + [_fullspec(D_IN, H)] * 4,
        out_specs=_rowspec(128),
        out_shape=jax.ShapeDtypeStruct((N + 16, 128), jnp.float32),
    )(x, degs, *W1s)

    acc1 = _run_agg(y1.reshape(16 * (N + 16), PW), edges, z8)

    y2 = pl.pallas_call(
        _k2_body,
        grid=(GRID,),
        in_specs=[_rowspec(128), _rowspec(128), _rowspec(H)]
        + [_fullspec(1, H)] * 4 + [_fullspec(H, H)] * 4,
        out_specs=_rowspec(128),
        out_shape=jax.ShapeDtypeStruct((N + 16, 128), jnp.float32),
    )(acc1, y1, degs, *b1s, *W2s)

    acc2 = _run_agg(y2.reshape(16 * (N + 16), PW), edges, z8)

    h2, stats = pl.pallas_call(
        _k3_body,
        grid=(GRID,),
        in_specs=[_rowspec(128), _rowspec(128), _rowspec(H)]
        + [_fullspec(1, H)] * 4,
        out_specs=[_rowspec(H), _fullspec(8, H)],
        out_shape=[jax.ShapeDtypeStruct((N, H), jnp.float32),
                   jax.ShapeDtypeStruct((8, H), jnp.float32)],
    )(acc2, y2, degs, *b2s)

    mean = stats[0] / N
    var = stats[1] / N - mean * mean
    scinv = (bn_g * lax.rsqrt(var + 1e-5)).reshape(1, H)

    out = pl.pallas_call(
        _k4_body,
        grid=(GRID,),
        in_specs=[_rowspec(H), _fullspec(1, H), _fullspec(1, H),
                  _fullspec(1, H), _fullspec(H, H), _fullspec(1, H),
                  _fullspec(H, 2), _fullspec(1, 2)],
        out_specs=_rowspec(2),
        out_shape=jax.ShapeDtypeStruct((N, 2), jnp.float32),
    )(h2, mean.reshape(1, H), scinv, bn_b.reshape(1, H), lin1_W,
      lin1_b.reshape(1, H), lin2_W, lin2_b.reshape(1, 2))
    return out


# speedup vs baseline: 1.0025x; 1.0025x over previous
import functools

import jax
import jax.numpy as jnp
from jax import lax
from jax.experimental import pallas as pl
from jax.experimental.pallas import tpu as pltpu
from jax.experimental.pallas import tpu_sc as plsc

N = 50000
D_IN = 128
H = 32

E = 800000
LANES = 256
NC = 2
NS = 16
RPT = 196
EROWS = RPT * NS
EPAD = EROWS * LANES - E

NACC = 50048
OPT = NACC // NS

P = 4
PW = H // P
CH = 1
NCH = RPT // CH

BLK = 2000
GRID = N // BLK


def _prep_edges(ei, r):
    src = jnp.concatenate(
        [ei[0] * 16 + 4 * r, jnp.full((EPAD,), 4 * r, jnp.int32)])
    dst = jnp.concatenate([ei[1], jnp.full((EPAD,), N, jnp.int32)])
    return src.reshape(EROWS, LANES), dst.reshape(EROWS, LANES)


@functools.cache
def _sc_mesh():
    return plsc.VectorSubcoreMesh(
        core_axis_name="c", subcore_axis_name="s",
        num_cores=NC, num_subcores=NS)


def _deg_body(edges, z8, o8, out, idx_v, ones_v, hist, sem):
    c = lax.axis_index("c")
    s = lax.axis_index("s")

    pltpu.sync_copy(o8, ones_v)

    def one_rel(r):
        pltpu.sync_copy(z8, hist.at[pl.ds(s * OPT, OPT)])
        pltpu.sync_copy(edges.at[4 + r].at[pl.ds(s * RPT, RPT)], idx_v)
        plsc.subcore_barrier()

        @pl.loop(0, RPT)
        def _(k):
            pltpu.async_copy(ones_v, hist.at[idx_v.at[k]], sem, add=True)

        @pl.loop(0, RPT)
        def _(k):
            pltpu.make_async_copy(ones_v, hist.at[idx_v.at[k]], sem).wait()

        plsc.subcore_barrier()
        pltpu.sync_copy(hist.at[pl.ds(s * OPT, OPT)],
                        out.at[pl.ds(s * OPT, OPT), pl.ds(8 * r, PW)])
        plsc.subcore_barrier()

    @pl.when(c == 0)
    def _():
        one_rel(0)
        one_rel(1)

    @pl.when(c == 1)
    def _():
        one_rel(2)
        one_rel(3)


@functools.cache
def _deg_kernel():
    return functools.partial(
        pl.kernel,
        out_type=jax.ShapeDtypeStruct((NACC, H), jnp.float32),
        mesh=_sc_mesh(),
        compiler_params=pltpu.CompilerParams(use_tc_tiling_on_sc=False),
        scratch_types=[
            pltpu.VMEM((RPT, LANES), jnp.int32),
            pltpu.VMEM((LANES, PW), jnp.float32),
            pltpu.VMEM_SHARED((NACC, PW), jnp.float32),
            pltpu.SemaphoreType.DMA,
        ],
    )(_deg_body)


def _run_deg(*args):
    return _deg_kernel()(*args)


def _agg_body(y8, edges, z8, out, isrc, idst, rows_a, rows_b, acc,
              gsem, ssem):
    c = lax.axis_index("c")
    s = lax.axis_index("s")

    def fire_g(yp, c0, buf):
        for i in range(CH):
            pltpu.async_copy(yp.at[isrc.at[c0 * CH + i]], buf.at[i], gsem)

    def wait_g(yp, c0, buf):
        for i in range(CH):
            pltpu.make_async_copy(
                yp.at[isrc.at[c0 * CH + i]], buf.at[i], gsem).wait()

    def fire_s(c0, buf):
        for i in range(CH):
            pltpu.async_copy(
                buf.at[i], acc.at[idst.at[c0 * CH + i]], ssem, add=True)

    def wait_s(c0, buf):
        for i in range(CH):
            pltpu.make_async_copy(
                buf.at[i], acc.at[idst.at[c0 * CH + i]], ssem).wait()

    def one_rel(r):
        pltpu.sync_copy(edges.at[r].at[pl.ds(s * RPT, RPT)], isrc)
        pltpu.sync_copy(edges.at[4 + r].at[pl.ds(s * RPT, RPT)], idst)
        for p in range(P):
            yp = y8.at[pl.ds(p, 16 * N)]
            pltpu.sync_copy(z8, acc.at[pl.ds(s * OPT, OPT)])
            plsc.subcore_barrier()

            fire_g(yp, 0, rows_a)

            @pl.loop(0, NCH, step=2)
            def _(c0):
                fire_g(yp, c0 + 1, rows_b)
                wait_g(yp, c0, rows_a)
                fire_s(c0, rows_a)
                wait_g(yp, c0 + 1, rows_b)
                fire_s(c0 + 1, rows_b)
                wait_s(c0, rows_a)

                @pl.when(c0 + 2 < NCH)
                def _():
                    fire_g(yp, c0 + 2, rows_a)

                wait_s(c0 + 1, rows_b)

            plsc.subcore_barrier()
            pltpu.sync_copy(
                acc.at[pl.ds(s * OPT, OPT)],
                out.at[pl.ds(s * OPT, OPT), pl.ds(32 * r + 8 * p, PW)])
            plsc.subcore_barrier()

    @pl.when(c == 0)
    def _():
        one_rel(0)
        one_rel(1)

    @pl.when(c == 1)
    def _():
        one_rel(2)
        one_rel(3)


@functools.cache
def _agg_kernel():
    return functools.partial(
        pl.kernel,
        out_type=jax.ShapeDtypeStruct((NACC, 128), jnp.float32),
        mesh=_sc_mesh(),
        compiler_params=pltpu.CompilerParams(use_tc_tiling_on_sc=False),
        scratch_types=[
            pltpu.VMEM((RPT, LANES), jnp.int32),
            pltpu.VMEM((RPT, LANES), jnp.int32),
            pltpu.VMEM((CH, LANES, PW), jnp.float32),
            pltpu.VMEM((CH, LANES, PW), jnp.float32),
            pltpu.VMEM_SHARED((NACC, PW), jnp.float32),
            pltpu.SemaphoreType.DMA,
            pltpu.SemaphoreType.DMA,
        ],
    )(_agg_body)


def _run_agg(*args):
    return _agg_kernel()(*args)


def _dis(deg_blk, r):
    return lax.rsqrt(deg_blk[:, 8 * r:8 * r + 1] + 1.0)


def _k1_body(x, g, w0, w1, w2, w3, yL):
    xb = x[...]
    gb = g[...]
    parts = []
    for r, w in enumerate((w0, w1, w2, w3)):
        parts.append(jnp.dot(xb, w[...],
                             preferred_element_type=jnp.float32) * _dis(gb, r))
    yL[...] = jnp.concatenate(parts, axis=1)


def _k2_body(aL, yL, g, b0, b1, b2, b3, w0, w1, w2, w3, oL):
    ab = aL[...]
    yb = yL[...]
    gb = g[...]
    ds_ = [_dis(gb, r) for r in range(4)]
    h = jnp.zeros((BLK, H), jnp.float32)
    for r, b in enumerate((b0, b1, b2, b3)):
        h = h + ds_[r] * (ab[:, 32 * r:32 * r + 32]
                          + yb[:, 32 * r:32 * r + 32]) + b[...]
    h = jnp.maximum(h, 0.0)
    parts = []
    for r, w in enumerate((w0, w1, w2, w3)):
        parts.append(jnp.dot(h, w[...],
                             preferred_element_type=jnp.float32) * ds_[r])
    oL[...] = jnp.concatenate(parts, axis=1)


def _k3_body(aL, yL, g, b0, b1, b2, b3, h2, stats):
    i = pl.program_id(0)
    ab = aL[...]
    yb = yL[...]
    gb = g[...]
    h = jnp.zeros((BLK, H), jnp.float32)
    for r, b in enumerate((b0, b1, b2, b3)):
        h = h + _dis(gb, r) * (ab[:, 32 * r:32 * r + 32]
                               + yb[:, 32 * r:32 * r + 32]) + b[...]
    h = jnp.maximum(h, 0.0)
    h2[...] = h

    part = jnp.concatenate(
        [jnp.sum(h, axis=0, keepdims=True),
         jnp.sum(h * h, axis=0, keepdims=True),
         jnp.zeros((6, H), jnp.float32)], axis=0)

    @pl.when(i == 0)
    def _():
        stats[...] = jnp.zeros((8, H), jnp.float32)

    stats[...] += part


def _k4_body(h2, mean, scinv, bnb, w1, b1, w2, b2, out):
    hb = (h2[...] - mean[...]) * scinv[...] + bnb[...]
    z = jnp.maximum(
        jnp.dot(hb, w1[...], preferred_element_type=jnp.float32) + b1[...],
        0.0)
    out[...] = jnp.dot(z, w2[...], preferred_element_type=jnp.float32) + b2[...]


def _rowspec(cols):
    return pl.BlockSpec((BLK, cols), lambda i: (i, 0))


def _fullspec(r, cols):
    return pl.BlockSpec((r, cols), lambda i: (0, 0))


def kernel(x, ei_fd, ei_fault, ei_rock, ei_geo, W1_fd, b1_fd, W2_fd, b2_fd,
           W1_fault, b1_fault, W2_fault, b2_fault, W1_rock, b1_rock, W2_rock,
           b2_rock, W1_geo, b1_geo, W2_geo, b2_geo, bn_g, bn_b, lin1_W,
           lin1_b, lin2_W, lin2_b):
    eis = [ei_fd, ei_fault, ei_rock, ei_geo]
    W1s = [W1_fd, W1_fault, W1_rock, W1_geo]
    b1s = [b1_fd.reshape(1, H), b1_fault.reshape(1, H),
           b1_rock.reshape(1, H), b1_geo.reshape(1, H)]
    W2s = [W2_fd, W2_fault, W2_rock, W2_geo]
    b2s = [b2_fd.reshape(1, H), b2_fault.reshape(1, H),
           b2_rock.reshape(1, H), b2_geo.reshape(1, H)]

    srcs, dsts = [], []
    for r, ei in enumerate(eis):
        s2, d2 = _prep_edges(ei, r)
        srcs.append(s2)
        dsts.append(d2)
    edges = jnp.stack(srcs + dsts)

    z8 = jnp.zeros((OPT, PW), jnp.float32)
    o8 = jnp.ones((LANES, PW), jnp.float32)

    degs = _run_deg(edges, z8, o8)

    y1 = pl.pallas_call(
        _k1_body,
        grid=(GRID,),
        in_specs=[_rowspec(D_IN), _rowspec(H)] + [_fullspec(D_IN, H)] * 4,
        out_specs=_rowspec(128),
        out_shape=jax.ShapeDtypeStruct((N + 16, 128), jnp.float32),
    )(x, degs, *W1s)

    acc1 = _run_agg(y1.reshape(16 * (N + 16), PW), edges, z8)

    y2 = pl.pallas_call(
        _k2_body,
        grid=(GRID,),
        in_specs=[_rowspec(128), _rowspec(128), _rowspec(H)]
        + [_fullspec(1, H)] * 4 + [_fullspec(H, H)] * 4,
        out_specs=_rowspec(128),
        out_shape=jax.ShapeDtypeStruct((N + 16, 128), jnp.float32),
    )(acc1, y1, degs, *b1s, *W2s)

    acc2 = _run_agg(y2.reshape(16 * (N + 16), PW), edges, z8)

    h2, stats = pl.pallas_call(
        _k3_body,
        grid=(GRID,),
        in_specs=[_rowspec(128), _rowspec(128), _rowspec(H)]
        + [_fullspec(1, H)] * 4,
        out_specs=[_rowspec(H), _fullspec(8, H)],
        out_shape=[jax.ShapeDtypeStruct((N, H), jnp.float32),
                   jax.ShapeDtypeStruct((8, H), jnp.float32)],
    )(acc2, y2, degs, *b2s)

    mean = stats[0] / N
    var = stats[1] / N - mean * mean
    scinv = (bn_g * lax.rsqrt(var + 1e-5)).reshape(1, H)

    out = pl.pallas_call(
        _k4_body,
        grid=(GRID,),
        in_specs=[_rowspec(H), _fullspec(1, H), _fullspec(1, H),
                  _fullspec(1, H), _fullspec(H, H), _fullspec(1, H),
                  _fullspec(H, 2), _fullspec(1, 2)],
        out_specs=_rowspec(2),
        out_shape=jax.ShapeDtypeStruct((N, 2), jnp.float32),
    )(h2, mean.reshape(1, H), scinv, bn_b.reshape(1, H), lin1_W,
      lin1_b.reshape(1, H), lin2_W, lin2_b.reshape(1, 2))
    return out


# speedup vs baseline: 1.0182x; 1.0157x over previous
import functools

import jax
import jax.numpy as jnp
from jax import lax
from jax.experimental import pallas as pl
from jax.experimental.pallas import tpu as pltpu
from jax.experimental.pallas import tpu_sc as plsc

N = 50000
D_IN = 128
H = 32

E = 800000
LANES = 128
NC = 2
NS = 16
RPT = 392
EROWS = RPT * NS
EPAD = EROWS * LANES - E

NACC = 50048
OPT = NACC // NS

P = 4
PW = H // P
CH = 2
NCH = RPT // CH

BLK = 2000
GRID = N // BLK


def _prep_edges(ei, r):
    src = jnp.concatenate(
        [ei[0] * 16 + 4 * r, jnp.full((EPAD,), 4 * r, jnp.int32)])
    dst = jnp.concatenate([ei[1], jnp.full((EPAD,), N, jnp.int32)])
    return src.reshape(EROWS, LANES), dst.reshape(EROWS, LANES)


@functools.cache
def _sc_mesh():
    return plsc.VectorSubcoreMesh(
        core_axis_name="c", subcore_axis_name="s",
        num_cores=NC, num_subcores=NS)


def _deg_body(edges, z8, o8, out, idx_v, ones_v, hist, sem):
    c = lax.axis_index("c")
    s = lax.axis_index("s")

    pltpu.sync_copy(o8, ones_v)

    def one_rel(r):
        pltpu.sync_copy(z8, hist.at[pl.ds(s * OPT, OPT)])
        pltpu.sync_copy(edges.at[4 + r].at[pl.ds(s * RPT, RPT)], idx_v)
        plsc.subcore_barrier()

        @pl.loop(0, RPT)
        def _(k):
            pltpu.async_copy(ones_v, hist.at[idx_v.at[k]], sem, add=True)

        @pl.loop(0, RPT)
        def _(k):
            pltpu.make_async_copy(ones_v, hist.at[idx_v.at[k]], sem).wait()

        plsc.subcore_barrier()
        pltpu.sync_copy(hist.at[pl.ds(s * OPT, OPT)],
                        out.at[pl.ds(s * OPT, OPT), pl.ds(8 * r, PW)])
        plsc.subcore_barrier()

    @pl.when(c == 0)
    def _():
        one_rel(0)
        one_rel(1)

    @pl.when(c == 1)
    def _():
        one_rel(2)
        one_rel(3)


@functools.cache
def _deg_kernel():
    return functools.partial(
        pl.kernel,
        out_type=jax.ShapeDtypeStruct((NACC, H), jnp.float32),
        mesh=_sc_mesh(),
        compiler_params=pltpu.CompilerParams(use_tc_tiling_on_sc=False),
        scratch_types=[
            pltpu.VMEM((RPT, LANES), jnp.int32),
            pltpu.VMEM((LANES, PW), jnp.float32),
            pltpu.VMEM_SHARED((NACC, PW), jnp.float32),
            pltpu.SemaphoreType.DMA,
        ],
    )(_deg_body)


def _run_deg(*args):
    return _deg_kernel()(*args)


def _agg_body(y8, edges, z8, out, isrc, idst, rows_a, rows_b, acc,
              gsem, ssem):
    c = lax.axis_index("c")
    s = lax.axis_index("s")

    def fire_g(yp, c0, buf):
        for i in range(CH):
            pltpu.async_copy(yp.at[isrc.at[c0 * CH + i]], buf.at[i], gsem)

    def wait_g(yp, c0, buf):
        for i in range(CH):
            pltpu.make_async_copy(
                yp.at[isrc.at[c0 * CH + i]], buf.at[i], gsem).wait()

    def fire_s(c0, buf):
        for i in range(CH):
            pltpu.async_copy(
                buf.at[i], acc.at[idst.at[c0 * CH + i]], ssem, add=True)

    def wait_s(c0, buf):
        for i in range(CH):
            pltpu.make_async_copy(
                buf.at[i], acc.at[idst.at[c0 * CH + i]], ssem).wait()

    def one_rel(r):
        pltpu.sync_copy(edges.at[r].at[pl.ds(s * RPT, RPT)], isrc)
        pltpu.sync_copy(edges.at[4 + r].at[pl.ds(s * RPT, RPT)], idst)
        for p in range(P):
            yp = y8.at[pl.ds(p, 16 * N)]
            pltpu.sync_copy(z8, acc.at[pl.ds(s * OPT, OPT)])
            plsc.subcore_barrier()

            fire_g(yp, 0, rows_a)

            @pl.loop(0, NCH, step=2)
            def _(c0):
                fire_g(yp, c0 + 1, rows_b)
                wait_g(yp, c0, rows_a)
                fire_s(c0, rows_a)
                wait_g(yp, c0 + 1, rows_b)
                fire_s(c0 + 1, rows_b)
                wait_s(c0, rows_a)

                @pl.when(c0 + 2 < NCH)
                def _():
                    fire_g(yp, c0 + 2, rows_a)

                wait_s(c0 + 1, rows_b)

            plsc.subcore_barrier()
            pltpu.sync_copy(
                acc.at[pl.ds(s * OPT, OPT)],
                out.at[pl.ds(s * OPT, OPT), pl.ds(32 * r + 8 * p, PW)])
            plsc.subcore_barrier()

    @pl.when(c == 0)
    def _():
        one_rel(0)
        one_rel(1)

    @pl.when(c == 1)
    def _():
        one_rel(2)
        one_rel(3)


@functools.cache
def _agg_kernel():
    return functools.partial(
        pl.kernel,
        out_type=jax.ShapeDtypeStruct((NACC, 128), jnp.float32),
        mesh=_sc_mesh(),
        compiler_params=pltpu.CompilerParams(use_tc_tiling_on_sc=False),
        scratch_types=[
            pltpu.VMEM((RPT, LANES), jnp.int32),
            pltpu.VMEM((RPT, LANES), jnp.int32),
            pltpu.VMEM((CH, LANES, PW), jnp.float32),
            pltpu.VMEM((CH, LANES, PW), jnp.float32),
            pltpu.VMEM_SHARED((NACC, PW), jnp.float32),
            pltpu.SemaphoreType.DMA,
            pltpu.SemaphoreType.DMA,
        ],
    )(_agg_body)


def _run_agg(*args):
    return _agg_kernel()(*args)


def _dis(deg_blk, r):
    return lax.rsqrt(deg_blk[:, 8 * r:8 * r + 1] + 1.0)


def _k1_body(x, g, w0, w1, w2, w3, yL):
    xb = x[...]
    gb = g[...]
    parts = []
    for r, w in enumerate((w0, w1, w2, w3)):
        parts.append(jnp.dot(xb, w[...],
                             preferred_element_type=jnp.float32) * _dis(gb, r))
    yL[...] = jnp.concatenate(parts, axis=1)


def _k2_body(aL, yL, g, b0, b1, b2, b3, w0, w1, w2, w3, oL):
    ab = aL[...]
    yb = yL[...]
    gb = g[...]
    ds_ = [_dis(gb, r) for r in range(4)]
    h = jnp.zeros((BLK, H), jnp.float32)
    for r, b in enumerate((b0, b1, b2, b3)):
        h = h + ds_[r] * (ab[:, 32 * r:32 * r + 32]
                          + yb[:, 32 * r:32 * r + 32]) + b[...]
    h = jnp.maximum(h, 0.0)
    parts = []
    for r, w in enumerate((w0, w1, w2, w3)):
        parts.append(jnp.dot(h, w[...],
                             preferred_element_type=jnp.float32) * ds_[r])
    oL[...] = jnp.concatenate(parts, axis=1)


def _k3_body(aL, yL, g, b0, b1, b2, b3, h2, stats):
    i = pl.program_id(0)
    ab = aL[...]
    yb = yL[...]
    gb = g[...]
    h = jnp.zeros((BLK, H), jnp.float32)
    for r, b in enumerate((b0, b1, b2, b3)):
        h = h + _dis(gb, r) * (ab[:, 32 * r:32 * r + 32]
                               + yb[:, 32 * r:32 * r + 32]) + b[...]
    h = jnp.maximum(h, 0.0)
    h2[...] = h

    part = jnp.concatenate(
        [jnp.sum(h, axis=0, keepdims=True),
         jnp.sum(h * h, axis=0, keepdims=True),
         jnp.zeros((6, H), jnp.float32)], axis=0)

    @pl.when(i == 0)
    def _():
        stats[...] = jnp.zeros((8, H), jnp.float32)

    stats[...] += part


def _k4_body(h2, mean, scinv, bnb, w1, b1, w2, b2, out):
    hb = (h2[...] - mean[...]) * scinv[...] + bnb[...]
    z = jnp.maximum(
        jnp.dot(hb, w1[...], preferred_element_type=jnp.float32) + b1[...],
        0.0)
    out[...] = jnp.dot(z, w2[...], preferred_element_type=jnp.float32) + b2[...]


def _rowspec(cols):
    return pl.BlockSpec((BLK, cols), lambda i: (i, 0))


def _fullspec(r, cols):
    return pl.BlockSpec((r, cols), lambda i: (0, 0))


def kernel(x, ei_fd, ei_fault, ei_rock, ei_geo, W1_fd, b1_fd, W2_fd, b2_fd,
           W1_fault, b1_fault, W2_fault, b2_fault, W1_rock, b1_rock, W2_rock,
           b2_rock, W1_geo, b1_geo, W2_geo, b2_geo, bn_g, bn_b, lin1_W,
           lin1_b, lin2_W, lin2_b):
    eis = [ei_fd, ei_fault, ei_rock, ei_geo]
    W1s = [W1_fd, W1_fault, W1_rock, W1_geo]
    b1s = [b1_fd.reshape(1, H), b1_fault.reshape(1, H),
           b1_rock.reshape(1, H), b1_geo.reshape(1, H)]
    W2s = [W2_fd, W2_fault, W2_rock, W2_geo]
    b2s = [b2_fd.reshape(1, H), b2_fault.reshape(1, H),
           b2_rock.reshape(1, H), b2_geo.reshape(1, H)]

    srcs, dsts = [], []
    for r, ei in enumerate(eis):
        s2, d2 = _prep_edges(ei, r)
        srcs.append(s2)
        dsts.append(d2)
    edges = jnp.stack(srcs + dsts)

    z8 = jnp.zeros((OPT, PW), jnp.float32)
    o8 = jnp.ones((LANES, PW), jnp.float32)

    degs = _run_deg(edges, z8, o8)

    y1 = pl.pallas_call(
        _k1_body,
        grid=(GRID,),
        in_specs=[_rowspec(D_IN), _rowspec(H)] + [_fullspec(D_IN, H)] * 4,
        out_specs=_rowspec(128),
        out_shape=jax.ShapeDtypeStruct((N + 16, 128), jnp.float32),
    )(x, degs, *W1s)

    acc1 = _run_agg(y1.reshape(16 * (N + 16), PW), edges, z8)

    y2 = pl.pallas_call(
        _k2_body,
        grid=(GRID,),
        in_specs=[_rowspec(128), _rowspec(128), _rowspec(H)]
        + [_fullspec(1, H)] * 4 + [_fullspec(H, H)] * 4,
        out_specs=_rowspec(128),
        out_shape=jax.ShapeDtypeStruct((N + 16, 128), jnp.float32),
    )(acc1, y1, degs, *b1s, *W2s)

    acc2 = _run_agg(y2.reshape(16 * (N + 16), PW), edges, z8)

    h2, stats = pl.pallas_call(
        _k3_body,
        grid=(GRID,),
        in_specs=[_rowspec(128), _rowspec(128), _rowspec(H)]
        + [_fullspec(1, H)] * 4,
        out_specs=[_rowspec(H), _fullspec(8, H)],
        out_shape=[jax.ShapeDtypeStruct((N, H), jnp.float32),
                   jax.ShapeDtypeStruct((8, H), jnp.float32)],
    )(acc2, y2, degs, *b2s)

    mean = stats[0] / N
    var = stats[1] / N - mean * mean
    scinv = (bn_g * lax.rsqrt(var + 1e-5)).reshape(1, H)

    out = pl.pallas_call(
        _k4_body,
        grid=(GRID,),
        in_specs=[_rowspec(H), _fullspec(1, H), _fullspec(1, H),
                  _fullspec(1, H), _fullspec(H, H), _fullspec(1, H),
                  _fullspec(H, 2), _fullspec(1, 2)],
        out_specs=_rowspec(2),
        out_shape=jax.ShapeDtypeStruct((N, 2), jnp.float32),
    )(h2, mean.reshape(1, H), scinv, bn_b.reshape(1, H), lin1_W,
      lin1_b.reshape(1, H), lin2_W, lin2_b.reshape(1, 2))
    return out


# speedup vs baseline: 1.0449x; 1.0262x over previous
import functools

import jax
import jax.numpy as jnp
from jax import lax
from jax.experimental import pallas as pl
from jax.experimental.pallas import tpu as pltpu
from jax.experimental.pallas import tpu_sc as plsc

N = 50000
D_IN = 128
H = 32

E = 800000
LANES = 128
NC = 2
NS = 16
RPT = 392
EROWS = RPT * NS
EPAD = EROWS * LANES - E

NACC = 50048
OPT = NACC // NS

P = 4
PW = H // P
CH = 2
NCH = RPT // CH

BLK = 2000
GRID = N // BLK


def _prep_edges(ei, r):
    src = jnp.concatenate(
        [ei[0] * 16 + 4 * r, jnp.full((EPAD,), 4 * r, jnp.int32)])
    dst = jnp.concatenate([ei[1], jnp.full((EPAD,), N, jnp.int32)])
    return src.reshape(EROWS, LANES), dst.reshape(EROWS, LANES)


@functools.cache
def _sc_mesh():
    return plsc.VectorSubcoreMesh(
        core_axis_name="c", subcore_axis_name="s",
        num_cores=NC, num_subcores=NS)


def _deg_body(edges, z8, o8, out, idx_v, ones_v, hist, sem):
    c = lax.axis_index("c")
    s = lax.axis_index("s")

    pltpu.sync_copy(o8, ones_v)

    def one_rel(r):
        pltpu.sync_copy(z8, hist.at[pl.ds(s * OPT, OPT)])
        pltpu.sync_copy(edges.at[4 + r].at[pl.ds(s * RPT, RPT)], idx_v)
        plsc.subcore_barrier()

        @pl.loop(0, RPT)
        def _(k):
            pltpu.async_copy(ones_v, hist.at[idx_v.at[k]], sem, add=True)

        @pl.loop(0, RPT)
        def _(k):
            pltpu.make_async_copy(ones_v, hist.at[idx_v.at[k]], sem).wait()

        plsc.subcore_barrier()
        pltpu.sync_copy(hist.at[pl.ds(s * OPT, OPT)],
                        out.at[pl.ds(s * OPT, OPT), pl.ds(8 * r, PW)])
        plsc.subcore_barrier()

    @pl.when(c == 0)
    def _():
        one_rel(0)
        one_rel(1)

    @pl.when(c == 1)
    def _():
        one_rel(2)
        one_rel(3)


@functools.cache
def _deg_kernel():
    return functools.partial(
        pl.kernel,
        out_type=jax.ShapeDtypeStruct((NACC, H), jnp.float32),
        mesh=_sc_mesh(),
        compiler_params=pltpu.CompilerParams(use_tc_tiling_on_sc=False),
        scratch_types=[
            pltpu.VMEM((RPT, LANES), jnp.int32),
            pltpu.VMEM((LANES, PW), jnp.float32),
            pltpu.VMEM_SHARED((NACC, PW), jnp.float32),
            pltpu.SemaphoreType.DMA,
        ],
    )(_deg_body)


def _run_deg(*args):
    return _deg_kernel()(*args)


def _agg_body(y8, edges, z8, out, isrc, idst, rows_a, rows_b, acc,
              gsem, ssem):
    c = lax.axis_index("c")
    s = lax.axis_index("s")

    def fire_g(yp, c0, buf):
        for i in range(CH):
            pltpu.async_copy(yp.at[isrc.at[c0 * CH + i]], buf.at[i], gsem)

    def wait_g(yp, c0, buf):
        for i in range(CH):
            pltpu.make_async_copy(
                yp.at[isrc.at[c0 * CH + i]], buf.at[i], gsem).wait()

    def fire_s(c0, buf):
        for i in range(CH):
            pltpu.async_copy(
                buf.at[i], acc.at[idst.at[c0 * CH + i]], ssem, add=True)

    def wait_s(c0, buf):
        for i in range(CH):
            pltpu.make_async_copy(
                buf.at[i], acc.at[idst.at[c0 * CH + i]], ssem).wait()

    def one_rel(r):
        pltpu.sync_copy(edges.at[r].at[pl.ds(s * RPT, RPT)], isrc)
        pltpu.sync_copy(edges.at[4 + r].at[pl.ds(s * RPT, RPT)], idst)
        for p in range(P):
            yp = y8.at[pl.ds(p, 16 * N)]
            fire_g(yp, 0, rows_a)
            pltpu.sync_copy(z8, acc.at[pl.ds(s * OPT, OPT)])
            plsc.subcore_barrier()

            @pl.loop(0, NCH, step=2)
            def _(c0):
                fire_g(yp, c0 + 1, rows_b)
                wait_g(yp, c0, rows_a)
                fire_s(c0, rows_a)
                wait_g(yp, c0 + 1, rows_b)
                fire_s(c0 + 1, rows_b)
                wait_s(c0, rows_a)

                @pl.when(c0 + 2 < NCH)
                def _():
                    fire_g(yp, c0 + 2, rows_a)

                wait_s(c0 + 1, rows_b)

            plsc.subcore_barrier()
            pltpu.sync_copy(
                acc.at[pl.ds(s * OPT, OPT)],
                out.at[pl.ds(s * OPT, OPT), pl.ds(32 * r + 8 * p, PW)])

    @pl.when(c == 0)
    def _():
        one_rel(0)
        one_rel(1)

    @pl.when(c == 1)
    def _():
        one_rel(2)
        one_rel(3)


@functools.cache
def _agg_kernel():
    return functools.partial(
        pl.kernel,
        out_type=jax.ShapeDtypeStruct((NACC, 128), jnp.float32),
        mesh=_sc_mesh(),
        compiler_params=pltpu.CompilerParams(use_tc_tiling_on_sc=False),
        scratch_types=[
            pltpu.VMEM((RPT, LANES), jnp.int32),
            pltpu.VMEM((RPT, LANES), jnp.int32),
            pltpu.VMEM((CH, LANES, PW), jnp.float32),
            pltpu.VMEM((CH, LANES, PW), jnp.float32),
            pltpu.VMEM_SHARED((NACC, PW), jnp.float32),
            pltpu.SemaphoreType.DMA,
            pltpu.SemaphoreType.DMA,
        ],
    )(_agg_body)


def _run_agg(*args):
    return _agg_kernel()(*args)


def _dis(deg_blk, r):
    return lax.rsqrt(deg_blk[:, 8 * r:8 * r + 1] + 1.0)


def _k1_body(x, g, w0, w1, w2, w3, yL):
    xb = x[...]
    gb = g[...]
    parts = []
    for r, w in enumerate((w0, w1, w2, w3)):
        parts.append(jnp.dot(xb, w[...],
                             preferred_element_type=jnp.float32) * _dis(gb, r))
    yL[...] = jnp.concatenate(parts, axis=1)


def _k2_body(aL, yL, g, b0, b1, b2, b3, w0, w1, w2, w3, oL):
    ab = aL[...]
    yb = yL[...]
    gb = g[...]
    ds_ = [_dis(gb, r) for r in range(4)]
    h = jnp.zeros((BLK, H), jnp.float32)
    for r, b in enumerate((b0, b1, b2, b3)):
        h = h + ds_[r] * (ab[:, 32 * r:32 * r + 32]
                          + yb[:, 32 * r:32 * r + 32]) + b[...]
    h = jnp.maximum(h, 0.0)
    parts = []
    for r, w in enumerate((w0, w1, w2, w3)):
        parts.append(jnp.dot(h, w[...],
                             preferred_element_type=jnp.float32) * ds_[r])
    oL[...] = jnp.concatenate(parts, axis=1)


def _k3_body(aL, yL, g, b0, b1, b2, b3, h2, stats):
    i = pl.program_id(0)
    ab = aL[...]
    yb = yL[...]
    gb = g[...]
    h = jnp.zeros((BLK, H), jnp.float32)
    for r, b in enumerate((b0, b1, b2, b3)):
        h = h + _dis(gb, r) * (ab[:, 32 * r:32 * r + 32]
                               + yb[:, 32 * r:32 * r + 32]) + b[...]
    h = jnp.maximum(h, 0.0)
    h2[...] = h

    part = jnp.concatenate(
        [jnp.sum(h, axis=0, keepdims=True),
         jnp.sum(h * h, axis=0, keepdims=True),
         jnp.zeros((6, H), jnp.float32)], axis=0)

    @pl.when(i == 0)
    def _():
        stats[...] = jnp.zeros((8, H), jnp.float32)

    stats[...] += part


def _k4_body(h2, mean, scinv, bnb, w1, b1, w2, b2, out):
    hb = (h2[...] - mean[...]) * scinv[...] + bnb[...]
    z = jnp.maximum(
        jnp.dot(hb, w1[...], preferred_element_type=jnp.float32) + b1[...],
        0.0)
    out[...] = jnp.dot(z, w2[...], preferred_element_type=jnp.float32) + b2[...]


def _rowspec(cols):
    return pl.BlockSpec((BLK, cols), lambda i: (i, 0))


def _fullspec(r, cols):
    return pl.BlockSpec((r, cols), lambda i: (0, 0))


def kernel(x, ei_fd, ei_fault, ei_rock, ei_geo, W1_fd, b1_fd, W2_fd, b2_fd,
           W1_fault, b1_fault, W2_fault, b2_fault, W1_rock, b1_rock, W2_rock,
           b2_rock, W1_geo, b1_geo, W2_geo, b2_geo, bn_g, bn_b, lin1_W,
           lin1_b, lin2_W, lin2_b):
    eis = [ei_fd, ei_fault, ei_rock, ei_geo]
    W1s = [W1_fd, W1_fault, W1_rock, W1_geo]
    b1s = [b1_fd.reshape(1, H), b1_fault.reshape(1, H),
           b1_rock.reshape(1, H), b1_geo.reshape(1, H)]
    W2s = [W2_fd, W2_fault, W2_rock, W2_geo]
    b2s = [b2_fd.reshape(1, H), b2_fault.reshape(1, H),
           b2_rock.reshape(1, H), b2_geo.reshape(1, H)]

    srcs, dsts = [], []
    for r, ei in enumerate(eis):
        s2, d2 = _prep_edges(ei, r)
        srcs.append(s2)
        dsts.append(d2)
    edges = jnp.stack(srcs + dsts)

    z8 = jnp.zeros((OPT, PW), jnp.float32)
    o8 = jnp.ones((LANES, PW), jnp.float32)

    degs = _run_deg(edges, z8, o8)

    y1 = pl.pallas_call(
        _k1_body,
        grid=(GRID,),
        in_specs=[_rowspec(D_IN), _rowspec(H)] + [_fullspec(D_IN, H)] * 4,
        out_specs=_rowspec(128),
        out_shape=jax.ShapeDtypeStruct((N + 16, 128), jnp.float32),
    )(x, degs, *W1s)

    acc1 = _run_agg(y1.reshape(16 * (N + 16), PW), edges, z8)

    y2 = pl.pallas_call(
        _k2_body,
        grid=(GRID,),
        in_specs=[_rowspec(128), _rowspec(128), _rowspec(H)]
        + [_fullspec(1, H)] * 4 + [_fullspec(H, H)] * 4,
        out_specs=_rowspec(128),
        out_shape=jax.ShapeDtypeStruct((N + 16, 128), jnp.float32),
    )(acc1, y1, degs, *b1s, *W2s)

    acc2 = _run_agg(y2.reshape(16 * (N + 16), PW), edges, z8)

    h2, stats = pl.pallas_call(
        _k3_body,
        grid=(GRID,),
        in_specs=[_rowspec(128), _rowspec(128), _rowspec(H)]
        + [_fullspec(1, H)] * 4,
        out_specs=[_rowspec(H), _fullspec(8, H)],
        out_shape=[jax.ShapeDtypeStruct((N, H), jnp.float32),
                   jax.ShapeDtypeStruct((8, H), jnp.float32)],
    )(acc2, y2, degs, *b2s)

    mean = stats[0] / N
    var = stats[1] / N - mean * mean
    scinv = (bn_g * lax.rsqrt(var + 1e-5)).reshape(1, H)

    out = pl.pallas_call(
        _k4_body,
        grid=(GRID,),
        in_specs=[_rowspec(H), _fullspec(1, H), _fullspec(1, H),
                  _fullspec(1, H), _fullspec(H, H), _fullspec(1, H),
                  _fullspec(H, 2), _fullspec(1, 2)],
        out_specs=_rowspec(2),
        out_shape=jax.ShapeDtypeStruct((N, 2), jnp.float32),
    )(h2, mean.reshape(1, H), scinv, bn_b.reshape(1, H), lin1_W,
      lin1_b.reshape(1, H), lin2_W, lin2_b.reshape(1, 2))
    return out
